# initial kernel scaffold (unmeasured)
import jax
import jax.numpy as jnp
from jax import lax
from jax.experimental import pallas as pl
from jax.experimental.pallas import tpu as pltpu

N_DEV = 4
M, K_SH, N = 4096, 1024, 8192
MB = 512
NB = M // MB
CHUNK = N // N_DEV
HALF = CHUNK // 2
HOPS = 2 * (N_DEV - 1)
TOTAL = NB * HOPS


def kernel(x, w_mat):
    x = x.astype(jnp.bfloat16)
    w_mat = w_mat.astype(jnp.bfloat16)

    def body(x_ref, w_ref, out_ref, acc_ref, tmpA, tmpB, stage_ref,
             send_semA, recv_semA, send_semB, recv_semB,
             creditA, creditB, out_sem):
        me = lax.axis_index("i")
        left = lax.rem(me + N_DEV - 1, N_DEV)
        right = lax.rem(me + 1, N_DEV)

        barrier = pltpu.get_barrier_semaphore()
        for nbr in (left, right):
            pl.semaphore_signal(barrier, inc=1, device_id=(nbr,),
                                device_id_type=pl.DeviceIdType.MESH)
        pl.semaphore_wait(barrier, 2)

        def colA(c):
            return c * CHUNK

        def colB(c):
            return c * CHUNK + HALF

        for b in range(NB):
            acc_ref[...] = jnp.dot(
                x_ref[b * MB:(b + 1) * MB, :], w_ref[...],
                preferred_element_type=jnp.bfloat16,
            )

            for h in range(HOPS):
                n = b * HOPS + h
                slot = n % 2
                if h < N_DEV - 1:
                    s = h
                    cA_send = lax.rem(me - s + 2 * N_DEV, N_DEV)
                    cA_recv = lax.rem(me - s - 1 + 2 * N_DEV, N_DEV)
                    cB_send = lax.rem(me + s, N_DEV)
                    cB_recv = lax.rem(me + s + 1, N_DEV)
                    dstA = tmpA.at[slot]
                    dstB = tmpB.at[slot]
                else:
                    s = h - (N_DEV - 1)
                    cA_send = lax.rem(me + 1 - s + 2 * N_DEV, N_DEV)
                    cB_send = lax.rem(me - 1 + s + 2 * N_DEV, N_DEV)
                    dstA = acc_ref.at[:, pl.ds(colA(cA_send), HALF)]
                    dstB = acc_ref.at[:, pl.ds(colB(cB_send), HALF)]

                srcA = acc_ref.at[:, pl.ds(colA(cA_send), HALF)]
                srcB = acc_ref.at[:, pl.ds(colB(cB_send), HALF)]

                if n >= 2:
                    pl.semaphore_wait(creditA, 1)
                    pl.semaphore_wait(creditB, 1)

                rdmaA = pltpu.make_async_remote_copy(
                    src_ref=srcA, dst_ref=dstA,
                    send_sem=send_semA.at[slot], recv_sem=recv_semA.at[slot],
                    device_id=(right,), device_id_type=pl.DeviceIdType.MESH,
                )
                rdmaB = pltpu.make_async_remote_copy(
                    src_ref=srcB, dst_ref=dstB,
                    send_sem=send_semB.at[slot], recv_sem=recv_semB.at[slot],
                    device_id=(left,), device_id_type=pl.DeviceIdType.MESH,
                )
                rdmaA.start()
                rdmaB.start()
                rdmaA.wait()
                rdmaB.wait()

                if h < N_DEV - 1:
                    a0 = colA(cA_recv)
                    acc_ref[:, pl.ds(a0, HALF)] = (
                        acc_ref[:, pl.ds(a0, HALF)] + tmpA[slot]
                    )
                    b0 = colB(cB_recv)
                    acc_ref[:, pl.ds(b0, HALF)] = (
                        acc_ref[:, pl.ds(b0, HALF)] + tmpB[slot]
                    )

                if n < TOTAL - 2:
                    pl.semaphore_signal(creditA, inc=1, device_id=(left,),
                                        device_id_type=pl.DeviceIdType.MESH)
                    pl.semaphore_signal(creditB, inc=1, device_id=(right,),
                                        device_id_type=pl.DeviceIdType.MESH)

            stage_ref[...] = jnp.maximum(acc_ref[...], 0).astype(jnp.float32)
            cp = pltpu.make_async_copy(
                stage_ref, out_ref.at[pl.ds(b * MB, MB), :], out_sem
            )
            cp.start()
            cp.wait()

    return pl.pallas_call(
        body,
        out_shape=jax.ShapeDtypeStruct((M, N), jnp.float32),
        in_specs=[
            pl.BlockSpec(memory_space=pltpu.VMEM),
            pl.BlockSpec(memory_space=pltpu.VMEM),
        ],
        out_specs=pl.BlockSpec(memory_space=pltpu.ANY),
        scratch_shapes=[
            pltpu.VMEM((MB, N), jnp.bfloat16),
            pltpu.VMEM((2, MB, HALF), jnp.bfloat16),
            pltpu.VMEM((2, MB, HALF), jnp.bfloat16),
            pltpu.VMEM((MB, N), jnp.float32),
            pltpu.SemaphoreType.DMA((2,)),
            pltpu.SemaphoreType.DMA((2,)),
            pltpu.SemaphoreType.DMA((2,)),
            pltpu.SemaphoreType.DMA((2,)),
            pltpu.SemaphoreType.REGULAR,
            pltpu.SemaphoreType.REGULAR,
            pltpu.SemaphoreType.DMA,
        ],
        compiler_params=pltpu.CompilerParams(
            collective_id=0,
            vmem_limit_bytes=100 * 1024 * 1024,
        ),
    )(x, w_mat)


# baseline (device time: 665483 ns/iter reference)
import jax
import jax.numpy as jnp
from jax import lax
from jax.experimental import pallas as pl
from jax.experimental.pallas import tpu as pltpu

N_DEV = 4
M, K_SH, N = 4096, 1024, 8192
MB = 512
NB = M // MB
CHUNK = N // N_DEV
HALF = CHUNK // 2
SUB = HALF // 2
N2 = N // 2
MICROS = 12
PAIRS = NB // 2


def kernel(x, w_mat):
    x = x.astype(jnp.bfloat16)
    w_mat = w_mat.astype(jnp.bfloat16)

    def body(x_ref, w_ref, out_ref, acc0, acc1, tmpA, tmpB,
             sendA, recvA, sendB, recvB, creditA, creditB, out_sems):
        me = lax.axis_index("i")
        left = lax.rem(me + N_DEV - 1, N_DEV)
        right = lax.rem(me + 1, N_DEV)

        barrier = pltpu.get_barrier_semaphore()
        for nbr in (left, right):
            pl.semaphore_signal(barrier, inc=1, device_id=(nbr,),
                                device_id_type=pl.DeviceIdType.MESH)
        pl.semaphore_wait(barrier, 2)

        def gemm_half(accX, beta, half):
            accX[:, half * N2:(half + 1) * N2] = jnp.dot(
                x_ref[pl.ds(beta * MB, MB), :],
                w_ref[:, half * N2:(half + 1) * N2],
                preferred_element_type=jnp.float32,
            ).astype(jnp.bfloat16)

        def wait_out(se):
            pltpu.make_async_copy(
                acc0, out_ref.at[pl.ds(0, MB), :], out_sems.at[se]
            ).wait()

        def out_copy(accX, beta, se):
            pltpu.make_async_copy(
                accX, out_ref.at[pl.ds(beta * MB, MB), :], out_sems.at[se]
            ).start()

        def dummy(tmp, send_sems, recv_sems, slot):
            return pltpu.make_async_remote_copy(
                src_ref=tmp.at[slot], dst_ref=tmp.at[slot],
                send_sem=send_sems.at[slot], recv_sem=recv_sems.at[slot],
                device_id=(right,), device_id_type=pl.DeviceIdType.MESH,
            )

        def micro_params(jjx):
            jm = jjx % MICROS
            s, sub = jm // 2, jm % 2
            off = sub * SUB
            accX = acc0 if jjx < MICROS else acc1
            if s < N_DEV - 1:
                cA_send = lax.rem(me - s + 2 * N_DEV, N_DEV)
                cA_recv = lax.rem(me - s - 1 + 2 * N_DEV, N_DEV)
                cB_send = lax.rem(me + s, N_DEV)
                cB_recv = lax.rem(me + s + 1, N_DEV)
                is_rs = True
            else:
                s2 = s - (N_DEV - 1)
                cA_send = lax.rem(me + 1 - s2 + 2 * N_DEV, N_DEV)
                cB_send = lax.rem(me - 1 + s2 + 2 * N_DEV, N_DEV)
                cA_recv = cB_recv = None
                is_rs = False
            srcA = cA_send * CHUNK + off
            srcB = cB_send * CHUNK + HALF + off
            rcvA = None if is_rs is False else cA_recv * CHUNK + off
            rcvB = None if is_rs is False else cB_recv * CHUNK + HALF + off
            return accX, srcA, srcB, rcvA, rcvB, is_rs

        def pair(t, carry):
            for jj in range(2 * MICROS):
                slot = jj % 2
                accS, sA, sB, _, _, is_rs = micro_params(jj)

                def acquire():
                    pl.semaphore_wait(creditA, 1)
                    pl.semaphore_wait(creditB, 1)
                    dummy(tmpA, sendA, recvA, slot).wait_send()
                    dummy(tmpB, sendB, recvB, slot).wait_send()

                if jj >= 2:
                    acquire()
                else:
                    pl.when(t > 0)(acquire)

                dstA = tmpA.at[slot] if is_rs else accS.at[:, pl.ds(sA, SUB)]
                dstB = tmpB.at[slot] if is_rs else accS.at[:, pl.ds(sB, SUB)]
                pltpu.make_async_remote_copy(
                    src_ref=accS.at[:, pl.ds(sA, SUB)], dst_ref=dstA,
                    send_sem=sendA.at[slot], recv_sem=recvA.at[slot],
                    device_id=(right,), device_id_type=pl.DeviceIdType.MESH,
                ).start()
                pltpu.make_async_remote_copy(
                    src_ref=accS.at[:, pl.ds(sB, SUB)], dst_ref=dstB,
                    send_sem=sendB.at[slot], recv_sem=recvB.at[slot],
                    device_id=(left,), device_id_type=pl.DeviceIdType.MESH,
                ).start()

                jp = (jj + 2 * MICROS - 1) % (2 * MICROS)
                slotp = jp % 2
                accP, _, _, rA, rB, p_rs = micro_params(jp)

                def consume():
                    dummy(tmpA, sendA, recvA, slotp).wait_recv()
                    dummy(tmpB, sendB, recvB, slotp).wait_recv()
                    if p_rs:
                        accP[:, pl.ds(rA, SUB)] = (
                            accP[:, pl.ds(rA, SUB)] + tmpA[slotp]
                        )
                        accP[:, pl.ds(rB, SUB)] = (
                            accP[:, pl.ds(rB, SUB)] + tmpB[slotp]
                        )

                def grant():
                    pl.semaphore_signal(creditA, inc=1, device_id=(left,),
                                        device_id_type=pl.DeviceIdType.MESH)
                    pl.semaphore_signal(creditB, inc=1, device_id=(right,),
                                        device_id_type=pl.DeviceIdType.MESH)

                if jj == 0:
                    def consume_grant():
                        consume()
                        grant()
                    pl.when(t > 0)(consume_grant)
                elif jj == 2 * MICROS - 1:
                    consume()
                    pl.when(t < PAIRS - 1)(grant)
                else:
                    consume()
                    grant()

                if jj == 0:
                    pl.when(t >= 1)(lambda: out_copy(acc1, 2 * t - 1, 1))
                elif jj == 4:
                    pl.when(t >= 1)(lambda: wait_out(1))
                    gemm_half(acc1, 2 * t + 1, 0)
                elif jj == 5:
                    gemm_half(acc1, 2 * t + 1, 1)
                elif jj == 12:
                    out_copy(acc0, 2 * t, 0)
                elif jj == 16:
                    def _g0():
                        wait_out(0)
                        gemm_half(acc0, 2 * t + 2, 0)
                    pl.when(t < PAIRS - 1)(_g0)
                elif jj == 17:
                    pl.when(t < PAIRS - 1)(
                        lambda: gemm_half(acc0, 2 * t + 2, 1)
                    )
            return carry

        gemm_half(acc0, 0, 0)
        gemm_half(acc0, 0, 1)
        lax.fori_loop(0, PAIRS, pair, 0)

        dummy(tmpA, sendA, recvA, 1).wait_recv()
        dummy(tmpB, sendB, recvB, 1).wait_recv()
        for slot in (0, 1):
            dummy(tmpA, sendA, recvA, slot).wait_send()
            dummy(tmpB, sendB, recvB, slot).wait_send()

        out_copy(acc1, NB - 1, 1)
        wait_out(0)
        wait_out(1)

    res = pl.pallas_call(
        body,
        out_shape=jax.ShapeDtypeStruct((M, N), jnp.bfloat16),
        in_specs=[
            pl.BlockSpec(memory_space=pltpu.VMEM),
            pl.BlockSpec(memory_space=pltpu.VMEM),
        ],
        out_specs=pl.BlockSpec(memory_space=pl.ANY),
        scratch_shapes=[
            pltpu.VMEM((MB, N), jnp.bfloat16),
            pltpu.VMEM((MB, N), jnp.bfloat16),
            pltpu.VMEM((2, MB, SUB), jnp.bfloat16),
            pltpu.VMEM((2, MB, SUB), jnp.bfloat16),
            pltpu.SemaphoreType.DMA((2,)),
            pltpu.SemaphoreType.DMA((2,)),
            pltpu.SemaphoreType.DMA((2,)),
            pltpu.SemaphoreType.DMA((2,)),
            pltpu.SemaphoreType.REGULAR,
            pltpu.SemaphoreType.REGULAR,
            pltpu.SemaphoreType.DMA((2,)),
        ],
        compiler_params=pltpu.CompilerParams(
            collective_id=0,
            vmem_limit_bytes=100 * 1024 * 1024,
        ),
    )(x, w_mat)
    return jnp.maximum(res, 0).astype(jnp.float32)
